# baseline (speedup 1.0000x reference)
import functools

import jax
import jax.numpy as jnp
from jax import lax
from jax.experimental import pallas as pl
from jax.experimental.pallas import tpu as pltpu
from jax.experimental.pallas import tpu_sc as plsc

_B, _N, _D0, _D1, _D2, _K = 128, 1024, 96, 64, 32, 512
_DP = 128
_BN = _B * _N
_R = 32
_T = 4096
_G = _BN // _T

_NC, _NS = 2, 16
_NW = _NC * _NS
_BPW = _BN // _NW
_CH = 512
_NCH = _BPW // _CH
_IG = 512
_NIG = _CH // _IG


def _table_body(cb, wd1, bd1, wd2, bd2, tbl_ref):
    th = jnp.maximum(
        jnp.dot(cb[...], wd1[...], preferred_element_type=jnp.float32)
        + bd1[...], 0.0)
    tbl = (jnp.dot(th, wd2[...], preferred_element_type=jnp.float32)
           + bd2[...])
    for r in range(_R):
        tbl_ref[pl.ds(r * _K, _K), :] = tbl


def _table(cb, W_d1, b_d1, W_d2, b_d2):
    return pl.pallas_call(
        _table_body,
        out_shape=jax.ShapeDtypeStruct((_R * _K, _D0), jnp.float32),
    )(cb, W_d1, b_d1, W_d2, b_d2)


def _encode_body(x_ref, we1, be1, we2, be2, cbt, idx_ref):
    h = jnp.maximum(
        jnp.dot(x_ref[...], we1[...], preferred_element_type=jnp.float32)
        + be1[...], 0.0)
    z = (jnp.dot(h, we2[...], preferred_element_type=jnp.float32)
         + be2[...])
    scores = jnp.dot(z, cbt[...], preferred_element_type=jnp.float32)
    z2 = jnp.sum(z * z, axis=1, keepdims=True)
    cbt_v = cbt[...]
    cb2 = jnp.sum(cbt_v * cbt_v, axis=0, keepdims=True)
    dist = (z2 - 2.0 * scores) + cb2
    mn = jnp.min(dist, axis=1, keepdims=True)
    ids = lax.broadcasted_iota(jnp.int32, (1, _K), 1).astype(jnp.float32)
    idx_f = jnp.min(jnp.where(dist <= mn, ids, float(_K)), axis=1,
                    keepdims=True)
    idx = idx_f.astype(jnp.int32)
    rep = (lax.broadcasted_iota(jnp.int32, idx.shape, 0) & (_R - 1)) * _K
    idx_ref[...] = idx + rep


def _encode(x2, W_e1, b_e1, W_e2, b_e2, cbt, off, n):
    full = lambda shape: pl.BlockSpec(shape, lambda i: (0,) * len(shape))
    return pl.pallas_call(
        _encode_body,
        grid=(n // _T,),
        in_specs=[
            pl.BlockSpec((_T, _D0), lambda i: (i + off, 0)),
            full((_D0, _D1)), full((1, _D1)),
            full((_D1, _D2)), full((1, _D2)),
            full((_D2, _K)),
        ],
        out_specs=pl.BlockSpec((_T, 1), lambda i: (i, 0)),
        out_shape=jax.ShapeDtypeStruct((n, 1), jnp.int32),
    )(x2, W_e1, b_e1, W_e2, b_e2, cbt)


@functools.cache
def _make_sc_gather(n):
    bpw = n // _NW
    nch = bpw // _CH

    @functools.partial(
        pl.kernel,
        mesh=plsc.VectorSubcoreMesh(core_axis_name="c", subcore_axis_name="s"),
        out_type=jax.ShapeDtypeStruct((n, _D0), jnp.float32),
        scratch_types=[
            pltpu.VMEM((bpw,), jnp.int32),
            pltpu.VMEM((_CH, _D0), jnp.float32),
            pltpu.VMEM((_CH, _D0), jnp.float32),
            pltpu.SemaphoreType.DMA,
            pltpu.SemaphoreType.DMA,
            pltpu.SemaphoreType.DMA,
            pltpu.SemaphoreType.DMA,
        ],
        compiler_params=pltpu.CompilerParams(use_tc_tiling_on_sc=False),
    )
    def _sc_gather(tbl_hbm, idx_hbm, out_hbm, idx_v, rows_a, rows_b,
                   gsem_a, gsem_b, wsem_a, wsem_b):
        wid = lax.axis_index("s") * _NC + lax.axis_index("c")
        base = wid * bpw
        pltpu.sync_copy(idx_hbm.at[pl.ds(base, bpw)], idx_v)

        rows = (rows_a, rows_b)
        gsem = (gsem_a, gsem_b)
        wsem = (wsem_a, wsem_b)

        def fire_gathers(k, s):
            hs = []
            for p in range(_NIG):
                off = k * _CH + p * _IG
                hs.append(pltpu.async_copy(
                    tbl_hbm.at[idx_v.at[pl.ds(off, _IG)]],
                    rows[s].at[pl.ds(p * _IG, _IG)], gsem[s]))
            return hs

        gh = [None, None]
        wh = [None, None]
        for k in range(nch):
            s = k % 2
            if wh[s] is not None:
                wh[s].wait()
            gh[s] = fire_gathers(k, s)
            if k > 0:
                o = 1 - s
                for h in gh[o]:
                    h.wait()
                wh[o] = pltpu.async_copy(
                    rows[o], out_hbm.at[pl.ds(base + (k - 1) * _CH, _CH)],
                    wsem[o])
        s_last = (nch - 1) % 2
        for h in gh[s_last]:
            h.wait()
        wh[s_last] = pltpu.async_copy(
            rows[s_last], out_hbm.at[pl.ds(base + (nch - 1) * _CH, _CH)],
            wsem[s_last])
        wh[0].wait()
        wh[1].wait()

    return _sc_gather


_P = 4


def kernel(x, W_e1, b_e1, W_e2, b_e2, codebook, W_d1, b_d1, W_d2, b_d2):
    x2 = x.reshape(_BN, _D0)
    part = _BN // _P
    tbl = _table(codebook, W_d1, b_d1.reshape(1, _D1),
                 W_d2, b_d2.reshape(1, _D0))
    gather = _make_sc_gather(part)
    outs = []
    for ph in range(_P):
        idx = _encode(x2, W_e1, b_e1.reshape(1, _D1), W_e2,
                      b_e2.reshape(1, _D2), codebook.T,
                      ph * (part // _T), part)
        outs.append(gather(tbl, idx.reshape(part)))
    out = jnp.concatenate(outs, axis=0)
    return out.reshape(_B, _N, _D0)


# speedup vs baseline: 1.0258x; 1.0258x over previous
import functools

import jax
import jax.numpy as jnp
from jax import lax
from jax.experimental import pallas as pl
from jax.experimental.pallas import tpu as pltpu
from jax.experimental.pallas import tpu_sc as plsc

_B, _N, _D0, _D1, _D2, _K = 128, 1024, 96, 64, 32, 512
_DP = 128
_BN = _B * _N
_R = 64
_T = 4096
_G = _BN // _T

_NC, _NS = 2, 16
_NW = _NC * _NS
_BPW = _BN // _NW
_CH = 512
_NCH = _BPW // _CH
_IG = 512
_NIG = _CH // _IG


def _table_body(cb, wd1, bd1, wd2, bd2, tbl_ref):
    th = jnp.maximum(
        jnp.dot(cb[...], wd1[...], preferred_element_type=jnp.float32)
        + bd1[...], 0.0)
    tbl = (jnp.dot(th, wd2[...], preferred_element_type=jnp.float32)
           + bd2[...])
    for r in range(_R):
        tbl_ref[pl.ds(r * _K, _K), :] = tbl


def _table(cb, W_d1, b_d1, W_d2, b_d2):
    return pl.pallas_call(
        _table_body,
        out_shape=jax.ShapeDtypeStruct((_R * _K, _D0), jnp.float32),
    )(cb, W_d1, b_d1, W_d2, b_d2)


def _encode_body(x_ref, we1, be1, we2, be2, cbt, idx_ref):
    h = jnp.maximum(
        jnp.dot(x_ref[...], we1[...], preferred_element_type=jnp.float32)
        + be1[...], 0.0)
    z = (jnp.dot(h, we2[...], preferred_element_type=jnp.float32)
         + be2[...])
    scores = jnp.dot(z, cbt[...], preferred_element_type=jnp.float32)
    z2 = jnp.sum(z * z, axis=1, keepdims=True)
    cbt_v = cbt[...]
    cb2 = jnp.sum(cbt_v * cbt_v, axis=0, keepdims=True)
    dist = (z2 - 2.0 * scores) + cb2
    mn = jnp.min(dist, axis=1, keepdims=True)
    ids = lax.broadcasted_iota(jnp.int32, (1, _K), 1).astype(jnp.float32)
    idx_f = jnp.min(jnp.where(dist <= mn, ids, float(_K)), axis=1,
                    keepdims=True)
    idx = idx_f.astype(jnp.int32)
    rep = (lax.broadcasted_iota(jnp.int32, idx.shape, 0) & (_R - 1)) * _K
    idx_ref[...] = idx + rep


def _encode(x2, W_e1, b_e1, W_e2, b_e2, cbt, off, n):
    full = lambda shape: pl.BlockSpec(shape, lambda i: (0,) * len(shape))
    return pl.pallas_call(
        _encode_body,
        grid=(n // _T,),
        in_specs=[
            pl.BlockSpec((_T, _D0), lambda i: (i + off, 0)),
            full((_D0, _D1)), full((1, _D1)),
            full((_D1, _D2)), full((1, _D2)),
            full((_D2, _K)),
        ],
        out_specs=pl.BlockSpec((_T, 1), lambda i: (i, 0)),
        out_shape=jax.ShapeDtypeStruct((n, 1), jnp.int32),
    )(x2, W_e1, b_e1, W_e2, b_e2, cbt)


@functools.cache
def _make_sc_gather(n):
    bpw = n // _NW
    nch = bpw // _CH

    @functools.partial(
        pl.kernel,
        mesh=plsc.VectorSubcoreMesh(core_axis_name="c", subcore_axis_name="s"),
        out_type=jax.ShapeDtypeStruct((n, _D0), jnp.float32),
        scratch_types=[
            pltpu.VMEM((bpw,), jnp.int32),
            pltpu.VMEM((_CH, _D0), jnp.float32),
            pltpu.VMEM((_CH, _D0), jnp.float32),
            pltpu.SemaphoreType.DMA,
            pltpu.SemaphoreType.DMA,
            pltpu.SemaphoreType.DMA,
            pltpu.SemaphoreType.DMA,
        ],
        compiler_params=pltpu.CompilerParams(use_tc_tiling_on_sc=False),
    )
    def _sc_gather(tbl_hbm, idx_hbm, out_hbm, idx_v, rows_a, rows_b,
                   gsem_a, gsem_b, wsem_a, wsem_b):
        wid = lax.axis_index("s") * _NC + lax.axis_index("c")
        base = wid * bpw
        pltpu.sync_copy(idx_hbm.at[pl.ds(base, bpw)], idx_v)

        rows = (rows_a, rows_b)
        gsem = (gsem_a, gsem_b)
        wsem = (wsem_a, wsem_b)

        def fire_gathers(k, s):
            hs = []
            for p in range(_NIG):
                off = k * _CH + p * _IG
                hs.append(pltpu.async_copy(
                    tbl_hbm.at[idx_v.at[pl.ds(off, _IG)]],
                    rows[s].at[pl.ds(p * _IG, _IG)], gsem[s]))
            return hs

        gh = [None, None]
        wh = [None, None]
        for k in range(nch):
            s = k % 2
            if wh[s] is not None:
                wh[s].wait()
            gh[s] = fire_gathers(k, s)
            if k > 0:
                o = 1 - s
                for h in gh[o]:
                    h.wait()
                wh[o] = pltpu.async_copy(
                    rows[o], out_hbm.at[pl.ds(base + (k - 1) * _CH, _CH)],
                    wsem[o])
        s_last = (nch - 1) % 2
        for h in gh[s_last]:
            h.wait()
        wh[s_last] = pltpu.async_copy(
            rows[s_last], out_hbm.at[pl.ds(base + (nch - 1) * _CH, _CH)],
            wsem[s_last])
        wh[0].wait()
        wh[1].wait()

    return _sc_gather


_P = 2


def kernel(x, W_e1, b_e1, W_e2, b_e2, codebook, W_d1, b_d1, W_d2, b_d2):
    x2 = x.reshape(_BN, _D0)
    part = _BN // _P
    tbl = _table(codebook, W_d1, b_d1.reshape(1, _D1),
                 W_d2, b_d2.reshape(1, _D0))
    gather = _make_sc_gather(part)
    outs = []
    for ph in range(_P):
        idx = _encode(x2, W_e1, b_e1.reshape(1, _D1), W_e2,
                      b_e2.reshape(1, _D2), codebook.T,
                      ph * (part // _T), part)
        outs.append(gather(tbl, idx.reshape(part)))
    out = jnp.concatenate(outs, axis=0)
    return out.reshape(_B, _N, _D0)


# speedup vs baseline: 1.0428x; 1.0165x over previous
import functools

import jax
import jax.numpy as jnp
from jax import lax
from jax.experimental import pallas as pl
from jax.experimental.pallas import tpu as pltpu
from jax.experimental.pallas import tpu_sc as plsc

_B, _N, _D0, _D1, _D2, _K = 128, 1024, 96, 64, 32, 512
_DP = 128
_BN = _B * _N
_R = 32
_T = 4096
_G = _BN // _T

_NC, _NS = 2, 16
_NW = _NC * _NS
_BPW = _BN // _NW
_CH = 256
_NCH = _BPW // _CH
_IG = 256
_NIG = _CH // _IG


def _table_body(cb, wd1, bd1, wd2, bd2, tbl_ref):
    th = jnp.maximum(
        jnp.dot(cb[...], wd1[...], preferred_element_type=jnp.float32)
        + bd1[...], 0.0)
    tbl = (jnp.dot(th, wd2[...], preferred_element_type=jnp.float32)
           + bd2[...])
    for r in range(_R):
        tbl_ref[pl.ds(r * _K, _K), :] = tbl


def _table(cb, W_d1, b_d1, W_d2, b_d2):
    return pl.pallas_call(
        _table_body,
        out_shape=jax.ShapeDtypeStruct((_R * _K, _DP), jnp.float32),
    )(cb, W_d1, b_d1, W_d2, b_d2)


def _encode_body(x_ref, we1, be1, we2, be2, cbt, idx_ref):
    h = jnp.maximum(
        jnp.dot(x_ref[...], we1[...], preferred_element_type=jnp.float32)
        + be1[...], 0.0)
    z = (jnp.dot(h, we2[...], preferred_element_type=jnp.float32)
         + be2[...])
    scores = jnp.dot(z, cbt[...], preferred_element_type=jnp.float32)
    z2 = jnp.sum(z * z, axis=1, keepdims=True)
    cbt_v = cbt[...]
    cb2 = jnp.sum(cbt_v * cbt_v, axis=0, keepdims=True)
    dist = (z2 - 2.0 * scores) + cb2
    mn = jnp.min(dist, axis=1, keepdims=True)
    ids = lax.broadcasted_iota(jnp.int32, (1, _K), 1).astype(jnp.float32)
    idx_f = jnp.min(jnp.where(dist <= mn, ids, float(_K)), axis=1,
                    keepdims=True)
    idx = idx_f.astype(jnp.int32)
    rep = (lax.broadcasted_iota(jnp.int32, idx.shape, 0) & (_R - 1)) * _K
    idx_ref[...] = idx + rep


def _encode(x2, W_e1, b_e1, W_e2, b_e2, cbt, off, n):
    full = lambda shape: pl.BlockSpec(shape, lambda i: (0,) * len(shape))
    return pl.pallas_call(
        _encode_body,
        grid=(n // _T,),
        in_specs=[
            pl.BlockSpec((_T, _D0), lambda i: (i + off, 0)),
            full((_D0, _D1)), full((1, _D1)),
            full((_D1, _D2)), full((1, _D2)),
            full((_D2, _K)),
        ],
        out_specs=pl.BlockSpec((_T, 1), lambda i: (i, 0)),
        out_shape=jax.ShapeDtypeStruct((n, 1), jnp.int32),
    )(x2, W_e1, b_e1, W_e2, b_e2, cbt)


@functools.cache
def _make_sc_gather(n):
    bpw = n // _NW
    nch = bpw // _CH

    @functools.partial(
        pl.kernel,
        mesh=plsc.VectorSubcoreMesh(core_axis_name="c", subcore_axis_name="s"),
        out_type=jax.ShapeDtypeStruct((n, _DP), jnp.float32),
        scratch_types=[
            pltpu.VMEM((bpw,), jnp.int32),
            pltpu.VMEM((_CH, _DP), jnp.float32),
            pltpu.VMEM((_CH, _DP), jnp.float32),
            pltpu.SemaphoreType.DMA,
            pltpu.SemaphoreType.DMA,
            pltpu.SemaphoreType.DMA,
            pltpu.SemaphoreType.DMA,
        ],
    )
    def _sc_gather(tbl_hbm, idx_hbm, out_hbm, idx_v, rows_a, rows_b,
                   gsem_a, gsem_b, wsem_a, wsem_b):
        wid = lax.axis_index("s") * _NC + lax.axis_index("c")
        base = wid * bpw
        pltpu.sync_copy(idx_hbm.at[pl.ds(base, bpw)], idx_v)

        rows = (rows_a, rows_b)
        gsem = (gsem_a, gsem_b)
        wsem = (wsem_a, wsem_b)

        def fire_gathers(k, s):
            hs = []
            for p in range(_NIG):
                off = k * _CH + p * _IG
                hs.append(pltpu.async_copy(
                    tbl_hbm.at[idx_v.at[pl.ds(off, _IG)]],
                    rows[s].at[pl.ds(p * _IG, _IG)], gsem[s]))
            return hs

        gh = [None, None]
        wh = [None, None]
        for k in range(nch):
            s = k % 2
            if wh[s] is not None:
                wh[s].wait()
            gh[s] = fire_gathers(k, s)
            if k > 0:
                o = 1 - s
                for h in gh[o]:
                    h.wait()
                wh[o] = pltpu.async_copy(
                    rows[o], out_hbm.at[pl.ds(base + (k - 1) * _CH, _CH)],
                    wsem[o])
        s_last = (nch - 1) % 2
        for h in gh[s_last]:
            h.wait()
        wh[s_last] = pltpu.async_copy(
            rows[s_last], out_hbm.at[pl.ds(base + (nch - 1) * _CH, _CH)],
            wsem[s_last])
        wh[0].wait()
        wh[1].wait()

    return _sc_gather


_P = 2


def kernel(x, W_e1, b_e1, W_e2, b_e2, codebook, W_d1, b_d1, W_d2, b_d2):
    x2 = x.reshape(_BN, _D0)
    part = _BN // _P
    w_d2p = jnp.pad(W_d2, ((0, 0), (0, _DP - _D0)))
    b_d2p = jnp.pad(b_d2, (0, _DP - _D0)).reshape(1, _DP)
    tbl = _table(codebook, W_d1, b_d1.reshape(1, _D1), w_d2p, b_d2p)
    gather = _make_sc_gather(part)
    outs = []
    for ph in range(_P):
        idx = _encode(x2, W_e1, b_e1.reshape(1, _D1), W_e2,
                      b_e2.reshape(1, _D2), codebook.T,
                      ph * (part // _T), part)
        outs.append(gather(tbl, idx.reshape(part)))
    out = jnp.concatenate(outs, axis=0)[:, :_D0]
    return out.reshape(_B, _N, _D0)
